# baseline (speedup 1.0000x reference)
import jax
import jax.numpy as jnp
from jax.experimental import pallas as pl
from jax.experimental.pallas import tpu as pltpu

_N = 10000
_F = 128
_BM = 400
_KMAIN = 9984
_KTAIL = _N - _KMAIN
_NSTEPS = _N // _BM


def _gnn_kernel(active_ref, features_ref, w_ref, adj_main_ref, adj_any_ref,
                out_ref, support_ref, tail_ref, sem_ref):
    i = pl.program_id(0)

    def _start_tail(c, slot):
        pltpu.make_async_copy(
            adj_any_ref.at[pl.ds(c * _BM, _BM), pl.ds(_KMAIN, _KTAIL)],
            tail_ref.at[slot],
            sem_ref.at[slot],
        ).start()

    @pl.when(i == 0)
    def _():
        s = jnp.dot(features_ref[...], w_ref[...],
                    preferred_element_type=jnp.float32)
        support_ref[...] = jnp.where(active_ref[0] != 0, jnp.tanh(s), s)
        _start_tail(0, 0)

    @pl.when(i + 1 < _NSTEPS)
    def _():
        _start_tail(i + 1, jax.lax.rem(i + 1, 2))

    slot = jax.lax.rem(i, 2)
    pltpu.make_async_copy(
        adj_any_ref.at[pl.ds(i * _BM, _BM), pl.ds(_KMAIN, _KTAIL)],
        tail_ref.at[slot],
        sem_ref.at[slot],
    ).wait()

    acc = jnp.dot(adj_main_ref[...], support_ref[pl.ds(0, _KMAIN), :],
                  preferred_element_type=jnp.float32)
    acc += jnp.dot(tail_ref[slot], support_ref[pl.ds(_KMAIN, _KTAIL), :],
                   preferred_element_type=jnp.float32)
    out_ref[...] = acc


def kernel(features, adj, W, active):
    active_arr = jnp.asarray(active, jnp.int32).reshape((1,))
    return pl.pallas_call(
        _gnn_kernel,
        grid_spec=pltpu.PrefetchScalarGridSpec(
            num_scalar_prefetch=1,
            grid=(_NSTEPS,),
            in_specs=[
                pl.BlockSpec((_N, _F), lambda i, a: (0, 0)),
                pl.BlockSpec((_F, _F), lambda i, a: (0, 0)),
                pl.BlockSpec((_BM, _KMAIN), lambda i, a: (i, 0)),
                pl.BlockSpec(memory_space=pl.ANY),
            ],
            out_specs=pl.BlockSpec((_BM, _F), lambda i, a: (i, 0)),
            scratch_shapes=[
                pltpu.VMEM((_N, _F), jnp.float32),
                pltpu.VMEM((2, _BM, _KTAIL), jnp.float32),
                pltpu.SemaphoreType.DMA((2,)),
            ],
        ),
        out_shape=jax.ShapeDtypeStruct((_N, _F), jnp.float32),
        compiler_params=pltpu.CompilerParams(
            dimension_semantics=("arbitrary",),
        ),
    )(active_arr, features, W, adj, adj)
